# baseline (speedup 1.0000x reference)
import jax
import jax.numpy as jnp
from jax import lax
from jax.experimental import pallas as pl
from jax.experimental.pallas import tpu as pltpu
from jax.experimental.pallas import tpu_sc as plsc

VOCAB = 1000000
EMB = 64
BATCH = 16384

NUM_CORES = 2
NUM_SUBCORES = 16
LANES = 16
NW = NUM_CORES * NUM_SUBCORES
B_PER_W = BATCH // NW
NBLK_FULL = VOCAB // 128
BLK_PER_W = -(-NBLK_FULL // NW)
TAIL_START = NBLK_FULL * 128
NCHUNK_IDX = BATCH // LANES
DUMMY_BASE = 2 * BATCH
N_STAGE = 4

_MESH = dict(core_axis_name="c", subcore_axis_name="s")
_PARAMS = dict(use_tc_tiling_on_sc=True, needs_layout_passes=False)


def _scan_body(u_hbm, v_hbm, table_t, tail_hbm, euv,
               su, sv, lists, x16, cnt, cur, blk, stage,
               sblk0, sblk1, *ssems):
    wid = lax.axis_index("s") * NUM_CORES + lax.axis_index("c")
    lane = lax.iota(jnp.int32, LANES)
    lane0 = lane == 0
    zero16 = jnp.zeros((LANES,), jnp.int32)

    lo_blk = jnp.minimum(wid * BLK_PER_W, NBLK_FULL - 220)
    nblk = jnp.minimum(BLK_PER_W, NBLK_FULL - lo_blk)
    lo = lo_blk * 128
    hi = jnp.where(wid == NW - 1, VOCAB, (lo_blk + nblk) * 128)

    bsems = (sblk0, sblk1)

    def fire_win(wi, buf):
        colsl = pl.ds(pl.multiple_of((lo_blk + 4 * wi) * 128, 128), 512)
        for a in range(8):
            pltpu.async_copy(table_t.at[pl.ds(8 * a, 8), colsl],
                             blk.at[pl.ds(buf * 64 + 8 * a, 8), :],
                             bsems[buf])

    def drain_win(buf):
        for a in range(8):
            pltpu.make_async_copy(
                table_t.at[pl.ds(8 * a, 8), pl.ds(0, 512)],
                blk.at[pl.ds(buf * 64 + 8 * a, 8), :],
                bsems[buf]).wait()

    fire_win(0, 0)

    pltpu.sync_copy(u_hbm, su)
    pltpu.sync_copy(v_hbm, sv)
    for k in range(256 // LANES):
        cnt[pl.ds(k * LANES, LANES)] = zero16

    def count_hits(ch, _):
        for src in (su, sv):
            x = src[pl.ds(ch * LANES, LANES)]
            m = (x >= lo) & (x < hi)
            n = plsc.all_reduce_population_count(m)[0]
            x16[...] = x

            def hit(j, mc):
                fv = plsc.all_reduce_ffs(mc)
                xf = plsc.load_gather(x16, [fv])
                b = (xf >> 7) - lo_blk
                c0 = plsc.load_gather(cnt, [b])
                plsc.store_scatter(cnt, [b], c0 + 1, mask=lane0)
                return mc & (lane != fv)

            lax.fori_loop(0, n, hit, m)
        return 0

    lax.fori_loop(0, NCHUNK_IDX, count_hits, 0)

    run = jnp.int32(0)
    for k in range(256 // LANES):
        vr = cnt[pl.ds(k * LANES, LANES)]
        incl = plsc.cumsum(vr)
        cur[pl.ds(k * LANES, LANES)] = incl - vr + run
        run = run + incl[15]

    def fill_lists(ch, _):
        for tbase, src in ((0, su), (BATCH, sv)):
            x = src[pl.ds(ch * LANES, LANES)]
            m = (x >= lo) & (x < hi)
            n = plsc.all_reduce_population_count(m)[0]
            x16[...] = x

            def hit(j, mc):
                fv = plsc.all_reduce_ffs(mc)
                xf = plsc.load_gather(x16, [fv])
                b = (xf >> 7) - lo_blk
                pos = plsc.load_gather(cur, [b])
                code = zero16 + (ch * LANES + tbase) + fv
                word = pos >> 1
                sh = (pos & 1) * 16
                g = plsc.load_gather(lists, [word])
                m16 = 65535 << sh
                plsc.store_scatter(lists, [word],
                                   (g & ~m16) | (code << sh), mask=lane0)
                plsc.store_scatter(cur, [b], pos + 1, mask=lane0)
                return mc & (lane != fv)

            lax.fori_loop(0, n, hit, m)
        return 0

    lax.fori_loop(0, NCHUNK_IDX, fill_lists, 0)

    for sp in range(N_STAGE):
        pltpu.async_copy(stage.at[sp],
                         euv.at[pl.ds((DUMMY_BASE + sp) * EMB, EMB)],
                         ssems[sp])

    def process_bucket(bkt, rowbase, coloff):
        nbv = plsc.load_gather(cnt, [zero16 + bkt])
        begv = plsc.load_gather(cur, [zero16 + bkt])
        n_b = nbv[0]
        beg = begv[0] - n_b

        def octet(oc, _):
            for sp in range(N_STAGE):
                t = oc * N_STAGE + sp

                @pl.when(t < n_b)
                def _():
                    j = beg + t
                    gw = plsc.load_gather(lists, [zero16 + (j >> 1)])
                    codev = (gw >> ((j & 1) * 16)) & 65535
                    isv = codev >= BATCH
                    i_v = codev & (BATCH - 1)
                    uval = plsc.load_gather(su, [i_v])
                    vval = plsc.load_gather(sv, [i_v])
                    idxv = jnp.where(isv, vval, uval)
                    col = idxv & 127
                    is_tail = idxv >= TAIL_START
                    toff = jnp.maximum(idxv - TAIL_START, 0) * EMB
                    pltpu.make_async_copy(
                        stage.at[sp],
                        euv.at[pl.ds((DUMMY_BASE + sp) * EMB, EMB)],
                        ssems[sp]).wait()
                    for cc in range(EMB // LANES):
                        bvals = plsc.load_gather(
                            blk, [rowbase + cc * LANES + lane, coloff + col])
                        toffc = toff + cc * LANES + lane
                        tvals = plsc.load_gather(
                            blk, [toffc >> 9, toffc & 511])
                        stage[sp, pl.ds(cc * LANES, LANES)] = jnp.where(
                            is_tail, tvals, bvals)
                    code0 = codev[0]
                    pltpu.async_copy(
                        stage.at[sp],
                        euv.at[pl.ds(code0 * EMB, EMB)],
                        ssems[sp])

            return 0

        lax.fori_loop(0, (n_b + N_STAGE - 1) // N_STAGE, octet, 0)

    nwin = (nblk + 3) // 4

    def win_pair(pr, _):
        for s in (0, 1):
            wi = pr * 2 + s

            @pl.when(wi < nwin)
            def _():
                @pl.when(wi + 1 < nwin)
                def _():
                    fire_win(wi + 1, s ^ 1)

                drain_win(s)
                for half in range(4):
                    kbl = wi * 4 + half

                    @pl.when(kbl < nblk)
                    def _():
                        process_bucket(kbl, zero16 + s * 64,
                                       zero16 + half * 128)

        return 0

    lax.fori_loop(0, (nwin + 1) // 2, win_pair, 0)

    @pl.when(wid == NW - 1)
    def _():
        pltpu.sync_copy(tail_hbm, blk.at[pl.ds(0, 8), :])
        process_bucket(nblk, zero16 + 64, zero16)

    for sp in range(N_STAGE):
        pltpu.make_async_copy(
            stage.at[sp],
            euv.at[pl.ds((DUMMY_BASE + sp) * EMB, EMB)],
            ssems[sp]).wait()


def _dot_body(euv, out_hbm, bu, bv, out_v):
    wid = lax.axis_index("s") * NUM_CORES + lax.axis_index("c")
    base = wid * B_PER_W
    pltpu.sync_copy(euv.at[pl.ds(base * EMB, B_PER_W * EMB)], bu)
    pltpu.sync_copy(euv.at[pl.ds((BATCH + base) * EMB, B_PER_W * EMB)], bv)

    lane = lax.iota(jnp.int32, LANES)
    perms = [lane ^ (1 << s) for s in range(4)]

    def group_body(g, _):
        acc = jnp.zeros((LANES,), jnp.float32)
        for l in range(LANES):
            r = g * LANES + l
            p = jnp.zeros((LANES,), jnp.float32)
            for cc in range(EMB // LANES):
                cu = bu[pl.ds(r * EMB + cc * LANES, LANES)]
                cv = bv[pl.ds(r * EMB + cc * LANES, LANES)]
                p = p + cu * cv
            for s in range(4):
                p = p + p.at[perms[s]].get(mode="promise_in_bounds")
            acc = jnp.where(lane == l, p, acc)
        out_v[pl.ds(g * LANES, LANES)] = acc
        return 0

    lax.fori_loop(0, B_PER_W // LANES, group_body, 0)
    pltpu.sync_copy(out_v, out_hbm.at[pl.ds(base, B_PER_W)])


@jax.jit
def _skipgram(u, v, emb_t, tail):
    mesh = plsc.VectorSubcoreMesh(**_MESH)
    euv = pl.kernel(
        _scan_body,
        out_type=jax.ShapeDtypeStruct(((2 * BATCH + N_STAGE) * EMB,),
                                      jnp.float32),
        mesh=mesh,
        compiler_params=pltpu.CompilerParams(**_PARAMS),
        scratch_types=[
            pltpu.VMEM((BATCH,), jnp.int32),
            pltpu.VMEM((BATCH,), jnp.int32),
            pltpu.VMEM((BATCH,), jnp.int32),
            pltpu.VMEM((LANES,), jnp.int32),
            pltpu.VMEM((256,), jnp.int32),
            pltpu.VMEM((256,), jnp.int32),
            pltpu.VMEM((128, 512), jnp.float32),
            pltpu.VMEM((N_STAGE, EMB), jnp.float32),
            pltpu.SemaphoreType.DMA,
            pltpu.SemaphoreType.DMA,
        ] + [pltpu.SemaphoreType.DMA] * N_STAGE,
    )(u, v, emb_t, tail)

    mesh2 = plsc.VectorSubcoreMesh(**_MESH)
    return pl.kernel(
        _dot_body,
        out_type=jax.ShapeDtypeStruct((BATCH,), jnp.float32),
        mesh=mesh2,
        compiler_params=pltpu.CompilerParams(**_PARAMS),
        scratch_types=[
            pltpu.VMEM((B_PER_W * EMB,), jnp.float32),
            pltpu.VMEM((B_PER_W * EMB,), jnp.float32),
            pltpu.VMEM((B_PER_W,), jnp.float32),
        ],
    )(euv)


def kernel(u, v, emb_weight):
    tail = emb_weight[TAIL_START:, :].reshape(8, 512)
    return _skipgram(u.astype(jnp.int32), v.astype(jnp.int32),
                     emb_weight.T, tail)


# speedup vs baseline: 1.0842x; 1.0842x over previous
import jax
import jax.numpy as jnp
from jax import lax
from jax.experimental import pallas as pl
from jax.experimental.pallas import tpu as pltpu
from jax.experimental.pallas import tpu_sc as plsc

VOCAB = 1000000
EMB = 64
BATCH = 16384

NUM_CORES = 2
NUM_SUBCORES = 16
LANES = 16
NW = NUM_CORES * NUM_SUBCORES
B_PER_W = BATCH // NW
NBLK_FULL = VOCAB // 128
BLK_PER_W = -(-NBLK_FULL // NW)
TAIL_START = NBLK_FULL * 128
NCHUNK_IDX = BATCH // LANES
DUMMY_BASE = 2 * BATCH
N_STAGE = 4

_MESH = dict(core_axis_name="c", subcore_axis_name="s")
_PARAMS = dict(use_tc_tiling_on_sc=True, needs_layout_passes=False)


def _scan_body(u_hbm, v_hbm, table_t, tail_hbm, euv,
               su, sv, lists, x16, cnt, cur, blk, tail_v, stage,
               sblk0, sblk1, *ssems):
    wid = lax.axis_index("s") * NUM_CORES + lax.axis_index("c")
    lane = lax.iota(jnp.int32, LANES)
    lane0 = lane == 0
    zero16 = jnp.zeros((LANES,), jnp.int32)

    lo_blk = jnp.minimum(wid * BLK_PER_W, NBLK_FULL - 218)
    nblk = jnp.minimum(BLK_PER_W, NBLK_FULL - lo_blk)
    lo = lo_blk * 128
    hi = jnp.where(wid == NW - 1, VOCAB, (lo_blk + nblk) * 128)

    bsems = (sblk0, sblk1)

    def fire_win(wi, buf):
        colsl = pl.ds(pl.multiple_of((lo_blk + 2 * wi) * 128, 128), 256)
        for a in range(8):
            pltpu.async_copy(table_t.at[pl.ds(8 * a, 8), colsl],
                             blk.at[pl.ds(buf * 64 + 8 * a, 8), :],
                             bsems[buf])

    def drain_win(buf):
        for a in range(8):
            pltpu.make_async_copy(
                table_t.at[pl.ds(8 * a, 8), pl.ds(0, 256)],
                blk.at[pl.ds(buf * 64 + 8 * a, 8), :],
                bsems[buf]).wait()

    fire_win(0, 0)

    pltpu.sync_copy(u_hbm, su)
    pltpu.sync_copy(v_hbm, sv)
    pltpu.sync_copy(tail_hbm, tail_v)
    for k in range(256 // LANES):
        cnt[pl.ds(k * LANES, LANES)] = zero16

    def count_hits(ch, _):
        for src in (su, sv):
            x = src[pl.ds(ch * LANES, LANES)]
            m = (x >= lo) & (x < hi)
            n = plsc.all_reduce_population_count(m)[0]
            x16[...] = x

            def hit(j, mc):
                fv = plsc.all_reduce_ffs(mc)
                xf = plsc.load_gather(x16, [fv])
                b = (xf >> 7) - lo_blk
                c0 = plsc.load_gather(cnt, [b])
                plsc.store_scatter(cnt, [b], c0 + 1, mask=lane0)
                return mc & (lane != fv)

            lax.fori_loop(0, n, hit, m)
        return 0

    lax.fori_loop(0, NCHUNK_IDX, count_hits, 0)

    run = jnp.int32(0)
    for k in range(256 // LANES):
        vr = cnt[pl.ds(k * LANES, LANES)]
        incl = plsc.cumsum(vr)
        cur[pl.ds(k * LANES, LANES)] = incl - vr + run
        run = run + incl[15]

    def fill_lists(ch, _):
        for tbase, src in ((0, su), (BATCH, sv)):
            x = src[pl.ds(ch * LANES, LANES)]
            m = (x >= lo) & (x < hi)
            n = plsc.all_reduce_population_count(m)[0]
            x16[...] = x

            def hit(j, mc):
                fv = plsc.all_reduce_ffs(mc)
                xf = plsc.load_gather(x16, [fv])
                b = (xf >> 7) - lo_blk
                pos = plsc.load_gather(cur, [b])
                code = zero16 + (ch * LANES + tbase) + fv
                plsc.store_scatter(lists, [pos], code, mask=lane0)
                plsc.store_scatter(cur, [b], pos + 1, mask=lane0)
                return mc & (lane != fv)

            lax.fori_loop(0, n, hit, m)
        return 0

    lax.fori_loop(0, NCHUNK_IDX, fill_lists, 0)

    for sp in range(N_STAGE):
        pltpu.async_copy(stage.at[sp],
                         euv.at[pl.ds((DUMMY_BASE + sp) * EMB, EMB)],
                         ssems[sp])

    def process_bucket(bkt, rowbase, coloff):
        nbv = plsc.load_gather(cnt, [zero16 + bkt])
        begv = plsc.load_gather(cur, [zero16 + bkt])
        n_b = nbv[0]
        beg = begv[0] - n_b

        def octet(oc, _):
            for sp in range(N_STAGE):
                t = oc * N_STAGE + sp

                @pl.when(t < n_b)
                def _():
                    codev = plsc.load_gather(lists, [zero16 + (beg + t)])
                    isv = codev >= BATCH
                    i_v = codev & (BATCH - 1)
                    uval = plsc.load_gather(su, [i_v])
                    vval = plsc.load_gather(sv, [i_v])
                    idxv = jnp.where(isv, vval, uval)
                    col = idxv & 127
                    is_tail = idxv >= TAIL_START
                    toff = jnp.maximum(idxv - TAIL_START, 0) * EMB
                    pltpu.make_async_copy(
                        stage.at[sp],
                        euv.at[pl.ds((DUMMY_BASE + sp) * EMB, EMB)],
                        ssems[sp]).wait()
                    for cc in range(EMB // LANES):
                        bvals = plsc.load_gather(
                            blk, [rowbase + cc * LANES + lane, coloff + col])
                        tvals = plsc.load_gather(
                            tail_v, [toff + cc * LANES + lane])
                        stage[sp, pl.ds(cc * LANES, LANES)] = jnp.where(
                            is_tail, tvals, bvals)
                    code0 = codev[0]
                    pltpu.async_copy(
                        stage.at[sp],
                        euv.at[pl.ds(code0 * EMB, EMB)],
                        ssems[sp])

            return 0

        lax.fori_loop(0, (n_b + N_STAGE - 1) // N_STAGE, octet, 0)

    nwin = (nblk + 1) // 2

    def win_pair(pr, _):
        for s in (0, 1):
            wi = pr * 2 + s

            @pl.when(wi < nwin)
            def _():
                @pl.when(wi + 1 < nwin)
                def _():
                    fire_win(wi + 1, s ^ 1)

                drain_win(s)
                for half in (0, 1):
                    kbl = wi * 2 + half

                    @pl.when(kbl < nblk)
                    def _():
                        process_bucket(kbl, zero16 + s * 64,
                                       zero16 + half * 128)

        return 0

    lax.fori_loop(0, (nwin + 1) // 2, win_pair, 0)

    @pl.when(wid == NW - 1)
    def _():
        process_bucket(nblk, zero16, zero16)

    for sp in range(N_STAGE):
        pltpu.make_async_copy(
            stage.at[sp],
            euv.at[pl.ds((DUMMY_BASE + sp) * EMB, EMB)],
            ssems[sp]).wait()


def _dot_body(euv, out_hbm, bu, bv, out_v):
    wid = lax.axis_index("s") * NUM_CORES + lax.axis_index("c")
    base = wid * B_PER_W
    pltpu.sync_copy(euv.at[pl.ds(base * EMB, B_PER_W * EMB)], bu)
    pltpu.sync_copy(euv.at[pl.ds((BATCH + base) * EMB, B_PER_W * EMB)], bv)

    lane = lax.iota(jnp.int32, LANES)
    perms = [lane ^ (1 << s) for s in range(4)]

    def group_body(g, _):
        acc = jnp.zeros((LANES,), jnp.float32)
        for l in range(LANES):
            r = g * LANES + l
            p = jnp.zeros((LANES,), jnp.float32)
            for cc in range(EMB // LANES):
                cu = bu[pl.ds(r * EMB + cc * LANES, LANES)]
                cv = bv[pl.ds(r * EMB + cc * LANES, LANES)]
                p = p + cu * cv
            for s in range(4):
                p = p + p.at[perms[s]].get(mode="promise_in_bounds")
            acc = jnp.where(lane == l, p, acc)
        out_v[pl.ds(g * LANES, LANES)] = acc
        return 0

    lax.fori_loop(0, B_PER_W // LANES, group_body, 0)
    pltpu.sync_copy(out_v, out_hbm.at[pl.ds(base, B_PER_W)])


@jax.jit
def _skipgram(u, v, emb_t, tail):
    mesh = plsc.VectorSubcoreMesh(**_MESH)
    euv = pl.kernel(
        _scan_body,
        out_type=jax.ShapeDtypeStruct(((2 * BATCH + N_STAGE) * EMB,),
                                      jnp.float32),
        mesh=mesh,
        compiler_params=pltpu.CompilerParams(**_PARAMS),
        scratch_types=[
            pltpu.VMEM((BATCH,), jnp.int32),
            pltpu.VMEM((BATCH,), jnp.int32),
            pltpu.VMEM((2 * BATCH,), jnp.int32),
            pltpu.VMEM((LANES,), jnp.int32),
            pltpu.VMEM((256,), jnp.int32),
            pltpu.VMEM((256,), jnp.int32),
            pltpu.VMEM((128, 256), jnp.float32),
            pltpu.VMEM((EMB * EMB,), jnp.float32),
            pltpu.VMEM((N_STAGE, EMB), jnp.float32),
            pltpu.SemaphoreType.DMA,
            pltpu.SemaphoreType.DMA,
        ] + [pltpu.SemaphoreType.DMA] * N_STAGE,
    )(u, v, emb_t, tail)

    mesh2 = plsc.VectorSubcoreMesh(**_MESH)
    return pl.kernel(
        _dot_body,
        out_type=jax.ShapeDtypeStruct((BATCH,), jnp.float32),
        mesh=mesh2,
        compiler_params=pltpu.CompilerParams(**_PARAMS),
        scratch_types=[
            pltpu.VMEM((B_PER_W * EMB,), jnp.float32),
            pltpu.VMEM((B_PER_W * EMB,), jnp.float32),
            pltpu.VMEM((B_PER_W,), jnp.float32),
        ],
    )(euv)


def kernel(u, v, emb_weight):
    tail = emb_weight[TAIL_START:, :].reshape(-1)
    return _skipgram(u.astype(jnp.int32), v.astype(jnp.int32),
                     emb_weight.T, tail)


# speedup vs baseline: 1.1017x; 1.0162x over previous
import jax
import jax.numpy as jnp
from jax import lax
from jax.experimental import pallas as pl
from jax.experimental.pallas import tpu as pltpu
from jax.experimental.pallas import tpu_sc as plsc

VOCAB = 1000000
EMB = 64
BATCH = 16384

NUM_CORES = 2
NUM_SUBCORES = 16
LANES = 16
NW = NUM_CORES * NUM_SUBCORES
B_PER_W = BATCH // NW
NBLK_FULL = VOCAB // 128
BLK_PER_W = -(-NBLK_FULL // NW)
TAIL_START = NBLK_FULL * 128
NCHUNK_IDX = BATCH // LANES
DUMMY_BASE = 2 * BATCH
N_STAGE = 2

_MESH = dict(core_axis_name="c", subcore_axis_name="s")
_PARAMS = dict(use_tc_tiling_on_sc=True, needs_layout_passes=False)


def _scan_body(u_hbm, v_hbm, table_t, tail_hbm, euv,
               su, sv, lists, x16, cnt, cur, blk, tail_v, stage,
               sblk0, sblk1, *ssems):
    wid = lax.axis_index("s") * NUM_CORES + lax.axis_index("c")
    lane = lax.iota(jnp.int32, LANES)
    lane0 = lane == 0
    zero16 = jnp.zeros((LANES,), jnp.int32)

    lo_blk = jnp.minimum(wid * BLK_PER_W, NBLK_FULL - 218)
    nblk = jnp.minimum(BLK_PER_W, NBLK_FULL - lo_blk)
    lo = lo_blk * 128
    hi = jnp.where(wid == NW - 1, VOCAB, (lo_blk + nblk) * 128)

    bsems = (sblk0, sblk1)

    def fire_win(wi, buf):
        colsl = pl.ds(pl.multiple_of((lo_blk + 2 * wi) * 128, 128), 256)
        for a in range(8):
            pltpu.async_copy(table_t.at[pl.ds(8 * a, 8), colsl],
                             blk.at[pl.ds(buf * 64 + 8 * a, 8), :],
                             bsems[buf])

    def drain_win(buf):
        for a in range(8):
            pltpu.make_async_copy(
                table_t.at[pl.ds(8 * a, 8), pl.ds(0, 256)],
                blk.at[pl.ds(buf * 64 + 8 * a, 8), :],
                bsems[buf]).wait()

    fire_win(0, 0)

    pltpu.sync_copy(u_hbm, su)
    pltpu.sync_copy(v_hbm, sv)
    pltpu.sync_copy(tail_hbm, tail_v)
    for k in range(256 // LANES):
        cnt[pl.ds(k * LANES, LANES)] = zero16

    def count_hits(ch, _):
        for src in (su, sv):
            x = src[pl.ds(ch * LANES, LANES)]
            m = (x >= lo) & (x < hi)
            n = plsc.all_reduce_population_count(m)[0]
            x16[...] = x

            def hit(j, mc):
                fv = plsc.all_reduce_ffs(mc)
                xf = plsc.load_gather(x16, [fv])
                b = (xf >> 7) - lo_blk
                c0 = plsc.load_gather(cnt, [b])
                plsc.store_scatter(cnt, [b], c0 + 1, mask=lane0)
                return mc & (lane != fv)

            lax.fori_loop(0, n, hit, m)
        return 0

    lax.fori_loop(0, NCHUNK_IDX, count_hits, 0)

    run = jnp.int32(0)
    for k in range(256 // LANES):
        vr = cnt[pl.ds(k * LANES, LANES)]
        incl = plsc.cumsum(vr)
        cur[pl.ds(k * LANES, LANES)] = incl - vr + run
        run = run + incl[15]

    def fill_lists(ch, _):
        for tbase, src in ((0, su), (BATCH, sv)):
            x = src[pl.ds(ch * LANES, LANES)]
            m = (x >= lo) & (x < hi)
            n = plsc.all_reduce_population_count(m)[0]
            x16[...] = x

            def hit(j, mc):
                fv = plsc.all_reduce_ffs(mc)
                xf = plsc.load_gather(x16, [fv])
                b = (xf >> 7) - lo_blk
                pos = plsc.load_gather(cur, [b])
                code = zero16 + (ch * LANES + tbase) + fv
                plsc.store_scatter(lists, [pos], code, mask=lane0)
                plsc.store_scatter(cur, [b], pos + 1, mask=lane0)
                return mc & (lane != fv)

            lax.fori_loop(0, n, hit, m)
        return 0

    lax.fori_loop(0, NCHUNK_IDX, fill_lists, 0)

    for sp in range(N_STAGE):
        pltpu.async_copy(stage.at[sp],
                         euv.at[pl.ds((DUMMY_BASE + sp) * EMB, EMB)],
                         ssems[sp])

    def process_bucket(bkt, rowbase, coloff):
        nbv = plsc.load_gather(cnt, [zero16 + bkt])
        begv = plsc.load_gather(cur, [zero16 + bkt])
        n_b = nbv[0]
        beg = begv[0] - n_b

        def octet(oc, _):
            for sp in range(N_STAGE):
                t = oc * N_STAGE + sp

                @pl.when(t < n_b)
                def _():
                    codev = plsc.load_gather(lists, [zero16 + (beg + t)])
                    isv = codev >= BATCH
                    i_v = codev & (BATCH - 1)
                    uval = plsc.load_gather(su, [i_v])
                    vval = plsc.load_gather(sv, [i_v])
                    idxv = jnp.where(isv, vval, uval)
                    col = idxv & 127
                    is_tail = idxv >= TAIL_START
                    toff = jnp.maximum(idxv - TAIL_START, 0) * EMB
                    pltpu.make_async_copy(
                        stage.at[sp],
                        euv.at[pl.ds((DUMMY_BASE + sp) * EMB, EMB)],
                        ssems[sp]).wait()
                    for cc in range(EMB // LANES):
                        bvals = plsc.load_gather(
                            blk, [rowbase + cc * LANES + lane, coloff + col])
                        tvals = plsc.load_gather(
                            tail_v, [toff + cc * LANES + lane])
                        stage[sp, pl.ds(cc * LANES, LANES)] = jnp.where(
                            is_tail, tvals, bvals)
                    code0 = codev[0]
                    pltpu.async_copy(
                        stage.at[sp],
                        euv.at[pl.ds(code0 * EMB, EMB)],
                        ssems[sp])

            return 0

        lax.fori_loop(0, (n_b + N_STAGE - 1) // N_STAGE, octet, 0)

    nwin = (nblk + 1) // 2

    def win_pair(pr, _):
        for s in (0, 1):
            wi = pr * 2 + s

            @pl.when(wi < nwin)
            def _():
                @pl.when(wi + 1 < nwin)
                def _():
                    fire_win(wi + 1, s ^ 1)

                drain_win(s)
                for half in (0, 1):
                    kbl = wi * 2 + half

                    @pl.when(kbl < nblk)
                    def _():
                        process_bucket(kbl, zero16 + s * 64,
                                       zero16 + half * 128)

        return 0

    lax.fori_loop(0, (nwin + 1) // 2, win_pair, 0)

    @pl.when(wid == NW - 1)
    def _():
        process_bucket(nblk, zero16, zero16)

    for sp in range(N_STAGE):
        pltpu.make_async_copy(
            stage.at[sp],
            euv.at[pl.ds((DUMMY_BASE + sp) * EMB, EMB)],
            ssems[sp]).wait()


def _dot_body(euv, out_hbm, bu, bv, out_v):
    wid = lax.axis_index("s") * NUM_CORES + lax.axis_index("c")
    base = wid * B_PER_W
    pltpu.sync_copy(euv.at[pl.ds(base * EMB, B_PER_W * EMB)], bu)
    pltpu.sync_copy(euv.at[pl.ds((BATCH + base) * EMB, B_PER_W * EMB)], bv)

    lane = lax.iota(jnp.int32, LANES)
    perms = [lane ^ (1 << s) for s in range(4)]

    def group_body(g, _):
        acc = jnp.zeros((LANES,), jnp.float32)
        for l in range(LANES):
            r = g * LANES + l
            p = jnp.zeros((LANES,), jnp.float32)
            for cc in range(EMB // LANES):
                cu = bu[pl.ds(r * EMB + cc * LANES, LANES)]
                cv = bv[pl.ds(r * EMB + cc * LANES, LANES)]
                p = p + cu * cv
            for s in range(4):
                p = p + p.at[perms[s]].get(mode="promise_in_bounds")
            acc = jnp.where(lane == l, p, acc)
        out_v[pl.ds(g * LANES, LANES)] = acc
        return 0

    lax.fori_loop(0, B_PER_W // LANES, group_body, 0)
    pltpu.sync_copy(out_v, out_hbm.at[pl.ds(base, B_PER_W)])


@jax.jit
def _skipgram(u, v, emb_t, tail):
    mesh = plsc.VectorSubcoreMesh(**_MESH)
    euv = pl.kernel(
        _scan_body,
        out_type=jax.ShapeDtypeStruct(((2 * BATCH + N_STAGE) * EMB,),
                                      jnp.float32),
        mesh=mesh,
        compiler_params=pltpu.CompilerParams(**_PARAMS),
        scratch_types=[
            pltpu.VMEM((BATCH,), jnp.int32),
            pltpu.VMEM((BATCH,), jnp.int32),
            pltpu.VMEM((2 * BATCH,), jnp.int32),
            pltpu.VMEM((LANES,), jnp.int32),
            pltpu.VMEM((256,), jnp.int32),
            pltpu.VMEM((256,), jnp.int32),
            pltpu.VMEM((128, 256), jnp.float32),
            pltpu.VMEM((EMB * EMB,), jnp.float32),
            pltpu.VMEM((N_STAGE, EMB), jnp.float32),
            pltpu.SemaphoreType.DMA,
            pltpu.SemaphoreType.DMA,
        ] + [pltpu.SemaphoreType.DMA] * N_STAGE,
    )(u, v, emb_t, tail)

    mesh2 = plsc.VectorSubcoreMesh(**_MESH)
    return pl.kernel(
        _dot_body,
        out_type=jax.ShapeDtypeStruct((BATCH,), jnp.float32),
        mesh=mesh2,
        compiler_params=pltpu.CompilerParams(**_PARAMS),
        scratch_types=[
            pltpu.VMEM((B_PER_W * EMB,), jnp.float32),
            pltpu.VMEM((B_PER_W * EMB,), jnp.float32),
            pltpu.VMEM((B_PER_W,), jnp.float32),
        ],
    )(euv)


def kernel(u, v, emb_weight):
    tail = emb_weight[TAIL_START:, :].reshape(-1)
    return _skipgram(u.astype(jnp.int32), v.astype(jnp.int32),
                     emb_weight.T, tail)
